# Initial kernel scaffold; baseline (speedup 1.0000x reference)
import jax
import jax.numpy as jnp
from jax.experimental import pallas as pl


def kernel(x, edge_index, batch, W_pre, b_pre, W_conv, b_conv, W_read, b_read):
    raise NotImplementedError("write your pallas kernel here")


# speedup vs baseline: 2.3594x; 2.3594x over previous
import functools

import jax
import jax.numpy as jnp
from jax.experimental import pallas as pl
from jax.experimental.pallas import tpu as pltpu

N = 10000
E = 320000
F_IN = 128
H = 300
G = 128

NP = 10240
HP = 160
HH = H // 2
BM = 512
NB = NP // BM
NT = 32


def _dinv_block(degp_blk):
    deg = 1.0 + jnp.sum(degp_blk, axis=0)
    return jax.lax.rsqrt(deg)


def _tc_a_body(x_ref, wp_ref, bp_ref, wc_ref, degp_ref, u_ref):
    dinv = _dinv_block(degp_ref[...])
    h = jax.nn.relu(
        jnp.dot(x_ref[...], wp_ref[...], preferred_element_type=jnp.float32)
        + bp_ref[...]
    )
    g = dinv[:, None] * h
    u = jnp.dot(g, wc_ref[...], preferred_element_type=jnp.float32)
    u_ref[0, :, 0:HH] = u[:, 0:HH]
    u_ref[0, :, HH:HP] = jnp.zeros((BM, HP - HH), jnp.float32)
    u_ref[1, :, 0:HH] = u[:, HH:H]
    u_ref[1, :, HH:HP] = jnp.zeros((BM, HP - HH), jnp.float32)


def _tc_a(x_p, W_pre, b_pre2, W_conv, degp):
    return pl.pallas_call(
        _tc_a_body,
        grid=(NB,),
        in_specs=[
            pl.BlockSpec((BM, F_IN), lambda i: (i, 0)),
            pl.BlockSpec((F_IN, H), lambda i: (0, 0)),
            pl.BlockSpec((1, H), lambda i: (0, 0)),
            pl.BlockSpec((H, H), lambda i: (0, 0)),
            pl.BlockSpec((NT, BM), lambda i: (0, i)),
        ],
        out_specs=pl.BlockSpec((2, BM, HP), lambda i: (0, i, 0)),
        out_shape=jax.ShapeDtypeStruct((2, NP, HP), jnp.float32),
    )(x_p, W_pre, b_pre2, W_conv, degp)


def _tc_c_body(s_ref, u_ref, degp_ref, bc_ref, wc_ref, un_ref):
    dinv = _dinv_block(degp_ref[...])
    su = jnp.concatenate(
        [
            s_ref[0, :, 0:HH] + u_ref[0, :, 0:HH],
            s_ref[1, :, 0:HH] + u_ref[1, :, 0:HH],
        ],
        axis=1,
    )
    h = jax.nn.relu(dinv[:, None] * su + bc_ref[...])
    g = dinv[:, None] * h
    un = jnp.dot(g, wc_ref[...], preferred_element_type=jnp.float32)
    un_ref[0, :, 0:HH] = un[:, 0:HH]
    un_ref[0, :, HH:HP] = jnp.zeros((BM, HP - HH), jnp.float32)
    un_ref[1, :, 0:HH] = un[:, HH:H]
    un_ref[1, :, HH:HP] = jnp.zeros((BM, HP - HH), jnp.float32)


def _tc_c(s, u, degp, b_conv2, W_conv):
    return pl.pallas_call(
        _tc_c_body,
        grid=(NB,),
        in_specs=[
            pl.BlockSpec((2, BM, HP), lambda i: (0, i, 0)),
            pl.BlockSpec((2, BM, HP), lambda i: (0, i, 0)),
            pl.BlockSpec((NT, BM), lambda i: (0, i)),
            pl.BlockSpec((1, H), lambda i: (0, 0)),
            pl.BlockSpec((H, H), lambda i: (0, 0)),
        ],
        out_specs=pl.BlockSpec((2, BM, HP), lambda i: (0, i, 0)),
        out_shape=jax.ShapeDtypeStruct((2, NP, HP), jnp.float32),
    )(s, u, degp, b_conv2, W_conv)


def _tc_d_body(s_ref, u_ref, degp_ref, bc_ref, wr_ref, batch_ref, out_ref, acc):
    i = pl.program_id(0)

    @pl.when(i == 0)
    def _init():
        acc[...] = jnp.zeros_like(acc)

    dinv = _dinv_block(degp_ref[...])
    su = jnp.concatenate(
        [
            s_ref[0, :, 0:HH] + u_ref[0, :, 0:HH],
            s_ref[1, :, 0:HH] + u_ref[1, :, 0:HH],
        ],
        axis=1,
    )
    h = jax.nn.relu(dinv[:, None] * su + bc_ref[...])
    r = jnp.sum(h * wr_ref[...], axis=1)
    ids = batch_ref[0, 0, :]
    oh = (ids[:, None] == jax.lax.broadcasted_iota(jnp.int32, (1, G), 1)).astype(
        jnp.float32
    )
    acc[0, :] += jnp.sum(oh * r[:, None], axis=0)
    acc[1, :] += jnp.sum(oh, axis=0)

    @pl.when(i == NB - 1)
    def _fin():
        out_ref[...] = (acc[0:1, :] / jnp.maximum(acc[1:2, :], 1.0))


def _tc_d(s, u, degp, b_conv2, W_read2, batch3):
    return pl.pallas_call(
        _tc_d_body,
        grid=(NB,),
        in_specs=[
            pl.BlockSpec((2, BM, HP), lambda i: (0, i, 0)),
            pl.BlockSpec((2, BM, HP), lambda i: (0, i, 0)),
            pl.BlockSpec((NT, BM), lambda i: (0, i)),
            pl.BlockSpec((1, H), lambda i: (0, 0)),
            pl.BlockSpec((1, H), lambda i: (0, 0)),
            pl.BlockSpec((1, 1, BM), lambda i: (i, 0, 0)),
        ],
        out_specs=pl.BlockSpec((1, G), lambda i: (0, 0)),
        out_shape=jax.ShapeDtypeStruct((1, G), jnp.float32),
        scratch_shapes=[pltpu.VMEM((2, G), jnp.float32)],
    )(s, u, degp, b_conv2, W_read2, batch3)


def _scatter_xla(u, src, dst):
    u_full = jnp.concatenate([u[0, :, 0:HH], u[1, :, 0:HH]], axis=1)
    msg = u_full[src]
    s_full = jax.ops.segment_sum(msg, dst, num_segments=NP)
    z = jnp.zeros((2, NP, HP), jnp.float32)
    z = z.at[0, :, 0:HH].set(s_full[:, 0:HH])
    z = z.at[1, :, 0:HH].set(s_full[:, HH:H])
    return z


def _deg_xla(dst):
    counts = jax.ops.segment_sum(jnp.ones((E,), jnp.float32), dst, num_segments=NP)
    return jnp.zeros((NT, NP), jnp.float32).at[0].set(counts)


def kernel(x, edge_index, batch, W_pre, b_pre, W_conv, b_conv, W_read, b_read):
    src = edge_index[0]
    dst = edge_index[1]
    x_p = jnp.pad(x, ((0, NP - N), (0, 0)))
    batch3 = jnp.pad(batch, (0, NP - N), constant_values=G).reshape(NB, 1, BM)
    b_pre2 = b_pre.reshape(1, H)
    b_conv2 = b_conv.reshape(1, H)
    W_read2 = W_read.reshape(1, H)

    degp = _deg_xla(dst)
    u = _tc_a(x_p, W_pre, b_pre2, W_conv, degp)
    for _ in range(2):
        s = _scatter_xla(u, src, dst)
        u = _tc_c(s, u, degp, b_conv2, W_conv)
    s = _scatter_xla(u, src, dst)
    out = _tc_d(s, u, degp, b_conv2, W_read2, batch3)
    return out.reshape(G) + b_read[0]


# speedup vs baseline: 5.6517x; 2.3954x over previous
import functools

import jax
import jax.numpy as jnp
from jax import lax
from jax.experimental import pallas as pl
from jax.experimental.pallas import tpu as pltpu
from jax.experimental.pallas import tpu_sc as plsc

N = 10000
E = 320000
F_IN = 128
H = 300
G = 128

NP = 10240
PW = 128
NPL = 3
BM = 512
NB = NP // BM
DW = 128


def _dinv_block(degp_blk):
    deg = 1.0 + degp_blk[0, :, 0] + degp_blk[1, :, 0]
    return jax.lax.rsqrt(deg)


def _write_planes(u_ref, u):
    u_ref[0] = u[:, 0:PW]
    u_ref[1] = u[:, PW : 2 * PW]
    u_ref[2] = jnp.concatenate(
        [u[:, 2 * PW : H], jnp.zeros((BM, NPL * PW - H), jnp.float32)], axis=1
    )


def _read_su(s_ref, u_ref):
    su = [s_ref[0, p] + s_ref[1, p] + u_ref[p] for p in range(NPL)]
    return jnp.concatenate(su, axis=1)[:, 0:H]


def _tc_a_body(x_ref, wp_ref, bp_ref, wc_ref, degp_ref, u_ref):
    dinv = _dinv_block(degp_ref[...])
    h = jax.nn.relu(
        jnp.dot(x_ref[...], wp_ref[...], preferred_element_type=jnp.float32)
        + bp_ref[...]
    )
    g = dinv[:, None] * h
    u = jnp.dot(g, wc_ref[...], preferred_element_type=jnp.float32)
    _write_planes(u_ref, u)


def _tc_a(x_p, W_pre, b_pre2, W_conv, degp):
    return pl.pallas_call(
        _tc_a_body,
        grid=(NB,),
        in_specs=[
            pl.BlockSpec((BM, F_IN), lambda i: (i, 0)),
            pl.BlockSpec((F_IN, H), lambda i: (0, 0)),
            pl.BlockSpec((1, H), lambda i: (0, 0)),
            pl.BlockSpec((H, H), lambda i: (0, 0)),
            pl.BlockSpec((2, BM, DW), lambda i: (0, i, 0)),
        ],
        out_specs=pl.BlockSpec((NPL, BM, PW), lambda i: (0, i, 0)),
        out_shape=jax.ShapeDtypeStruct((NPL, NP, PW), jnp.float32),
    )(x_p, W_pre, b_pre2, W_conv, degp)


def _tc_c_body(s_ref, u_ref, degp_ref, bc_ref, wc_ref, un_ref):
    dinv = _dinv_block(degp_ref[...])
    su = _read_su(s_ref, u_ref)
    h = jax.nn.relu(dinv[:, None] * su + bc_ref[...])
    g = dinv[:, None] * h
    un = jnp.dot(g, wc_ref[...], preferred_element_type=jnp.float32)
    _write_planes(un_ref, un)


def _tc_c(s, u, degp, b_conv2, W_conv):
    return pl.pallas_call(
        _tc_c_body,
        grid=(NB,),
        in_specs=[
            pl.BlockSpec((2, NPL, BM, PW), lambda i: (0, 0, i, 0)),
            pl.BlockSpec((NPL, BM, PW), lambda i: (0, i, 0)),
            pl.BlockSpec((2, BM, DW), lambda i: (0, i, 0)),
            pl.BlockSpec((1, H), lambda i: (0, 0)),
            pl.BlockSpec((H, H), lambda i: (0, 0)),
        ],
        out_specs=pl.BlockSpec((NPL, BM, PW), lambda i: (0, i, 0)),
        out_shape=jax.ShapeDtypeStruct((NPL, NP, PW), jnp.float32),
    )(s, u, degp, b_conv2, W_conv)


def _tc_d_body(s_ref, u_ref, degp_ref, bc_ref, wr_ref, batch_ref, out_ref, acc):
    i = pl.program_id(0)

    @pl.when(i == 0)
    def _init():
        acc[...] = jnp.zeros_like(acc)

    dinv = _dinv_block(degp_ref[...])
    su = _read_su(s_ref, u_ref)
    h = jax.nn.relu(dinv[:, None] * su + bc_ref[...])
    r = jnp.sum(h * wr_ref[...], axis=1)
    ids = batch_ref[0, 0, :]
    oh = (ids[:, None] == jax.lax.broadcasted_iota(jnp.int32, (1, G), 1)).astype(
        jnp.float32
    )
    acc[0, :] += jnp.sum(oh * r[:, None], axis=0)
    acc[1, :] += jnp.sum(oh, axis=0)

    @pl.when(i == NB - 1)
    def _fin():
        out_ref[...] = (acc[0:1, :] / jnp.maximum(acc[1:2, :], 1.0))


def _tc_d(s, u, degp, b_conv2, W_read2, batch3):
    return pl.pallas_call(
        _tc_d_body,
        grid=(NB,),
        in_specs=[
            pl.BlockSpec((2, NPL, BM, PW), lambda i: (0, 0, i, 0)),
            pl.BlockSpec((NPL, BM, PW), lambda i: (0, i, 0)),
            pl.BlockSpec((2, BM, DW), lambda i: (0, i, 0)),
            pl.BlockSpec((1, H), lambda i: (0, 0)),
            pl.BlockSpec((1, H), lambda i: (0, 0)),
            pl.BlockSpec((1, 1, BM), lambda i: (i, 0, 0)),
        ],
        out_specs=pl.BlockSpec((1, G), lambda i: (0, 0)),
        out_shape=jax.ShapeDtypeStruct((1, G), jnp.float32),
        scratch_shapes=[pltpu.VMEM((2, G), jnp.float32)],
    )(s, u, degp, b_conv2, W_read2, batch3)


_MESH = plsc.VectorSubcoreMesh(core_axis_name="c", subcore_axis_name="s")
NSUB = 16
RPT = NP // NSUB
CH = 80
EC = E // (2 * NSUB)
NCH = EC // CH


@functools.partial(
    pl.kernel,
    mesh=_MESH,
    out_type=jax.ShapeDtypeStruct((2 * NP, DW), jnp.float32),
    scratch_types=[
        pltpu.VMEM((CH,), jnp.int32),
        pltpu.VMEM((CH, DW), jnp.float32),
        pltpu.VMEM((CH, DW), jnp.float32),
        pltpu.VMEM_SHARED((NP, DW), jnp.float32),
    ],
)
def _sc_deg(dst_hbm, out_hbm, dstv, ones_v, zeros_v, acc_sh):
    c = lax.axis_index("c")
    t = lax.axis_index("s")
    wid = c * NSUB + t

    def _fill(r, carry):
        ones_v[r, :] = jnp.ones((DW,), jnp.float32)
        zeros_v[r, :] = jnp.zeros((DW,), jnp.float32)
        return carry

    lax.fori_loop(0, CH, _fill, 0)
    for b in range(RPT // CH):
        pltpu.sync_copy(zeros_v, acc_sh.at[pl.ds(t * RPT + b * CH, CH)])
    plsc.subcore_barrier()

    def _step(k, carry):
        base = wid * EC + k * CH
        pltpu.sync_copy(dst_hbm.at[pl.ds(base, CH)], dstv)
        pltpu.sync_copy(ones_v, acc_sh.at[dstv], add=True)
        return carry

    lax.fori_loop(0, NCH, _step, 0)
    plsc.subcore_barrier()
    pltpu.sync_copy(acc_sh.at[pl.ds(t * RPT, RPT)],
                    out_hbm.at[pl.ds(c * NP + t * RPT, RPT)])


@functools.partial(
    pl.kernel,
    mesh=_MESH,
    out_type=jax.ShapeDtypeStruct((2 * NPL * NP, PW), jnp.float32),
    scratch_types=[
        pltpu.VMEM((CH,), jnp.int32),
        pltpu.VMEM((CH,), jnp.int32),
        pltpu.VMEM((CH, PW), jnp.float32),
        pltpu.VMEM_SHARED((NP, PW), jnp.float32),
        pltpu.SemaphoreType.DMA,
    ],
)
def _sc_scatter(u_hbm, src_hbm, dst_hbm, out_hbm, srcv, dstv, stag, acc_sh, sem):
    c = lax.axis_index("c")
    t = lax.axis_index("s")

    def _zrow(r, carry):
        for j in range(PW // 16):
            stag[r, pl.ds(j * 16, 16)] = jnp.zeros((16,), jnp.float32)
        return carry

    lax.fori_loop(0, CH, _zrow, 0)

    for p in range(NPL):
        for b in range(RPT // CH):
            pltpu.sync_copy(stag, acc_sh.at[pl.ds(t * RPT + b * CH, CH)])
        plsc.subcore_barrier()

        offv = jnp.broadcast_to(p * NP, (16,)).astype(jnp.int32)

        def _step(k, carry):
            base = (c * NSUB + t) * EC + k * CH
            pltpu.sync_copy(src_hbm.at[pl.ds(base, CH)], srcv)
            pltpu.sync_copy(dst_hbm.at[pl.ds(base, CH)], dstv)
            for j in range(CH // 16):
                srcv[pl.ds(j * 16, 16)] = srcv[pl.ds(j * 16, 16)] + offv
            pltpu.async_copy(u_hbm.at[srcv], stag, sem).wait()
            pltpu.sync_copy(stag, acc_sh.at[dstv], add=True)
            return carry

        lax.fori_loop(0, NCH, _step, 0)
        plsc.subcore_barrier()
        pltpu.sync_copy(
            acc_sh.at[pl.ds(t * RPT, RPT)],
            out_hbm.at[pl.ds((c * NPL + p) * NP + t * RPT, RPT)],
        )
        lax.fori_loop(0, CH, _zrow, 0)


def kernel(x, edge_index, batch, W_pre, b_pre, W_conv, b_conv, W_read, b_read):
    src = edge_index[0]
    dst = edge_index[1]
    x_p = jnp.pad(x, ((0, NP - N), (0, 0)))
    batch3 = jnp.pad(batch, (0, NP - N), constant_values=G).reshape(NB, 1, BM)
    b_pre2 = b_pre.reshape(1, H)
    b_conv2 = b_conv.reshape(1, H)
    W_read2 = W_read.reshape(1, H)

    degp = _sc_deg(dst).reshape(2, NP, DW)
    u = _tc_a(x_p, W_pre, b_pre2, W_conv, degp)
    for _ in range(2):
        s = _sc_scatter(u.reshape(NPL * NP, PW), src, dst).reshape(2, NPL, NP, PW)
        u = _tc_c(s, u, degp, b_conv2, W_conv)
    s = _sc_scatter(u.reshape(NPL * NP, PW), src, dst).reshape(2, NPL, NP, PW)
    out = _tc_d(s, u, degp, b_conv2, W_read2, batch3)
    return out.reshape(G) + b_read[0]
